# baseline (speedup 1.0000x reference)
import functools

import jax
import jax.numpy as jnp
from jax import lax
from jax.experimental import pallas as pl
from jax.experimental.pallas import tpu as pltpu
from jax.experimental.pallas import tpu_sc as plsc

N_NODES = 10000
N_PAD = 10240
E_EDGES = 320000
NC, NS = 2, 16
NW = NC * NS
CHUNK = 128
EPW = E_EDGES // NW
K_CHUNKS = -(-EPW // CHUNK)
EPW_PAD = K_CHUNKS * CHUNK
RPT = N_PAD // NS
BT = 1024

_MESH = plsc.VectorSubcoreMesh(core_axis_name="c", subcore_axis_name="s")
_SC_PARAMS = pltpu.CompilerParams(use_tc_tiling_on_sc=False)


@functools.partial(
    pl.kernel,
    out_type=jax.ShapeDtypeStruct((NC, N_PAD, 16), jnp.float32),
    mesh=_MESH,
    scratch_types=[
        pltpu.VMEM((K_CHUNKS, CHUNK), jnp.int32),
        pltpu.VMEM((CHUNK, 16), jnp.float32),
        pltpu.VMEM_SHARED((N_PAD, 16), jnp.float32),
        pltpu.SemaphoreType.DMA,
    ],
    compiler_params=_SC_PARAMS,
)
def _deg_kernel(dst_hbm, zrows_hbm, out_hbm, idx_v, ones_v, acc_sh, sem):
    c = lax.axis_index("c")
    s = lax.axis_index("s")
    wid = c * NS + s
    lane = lax.iota(jnp.int32, 16)
    unit = jnp.where(lane == 0, jnp.float32(1.0), jnp.float32(0.0))

    @pl.loop(0, CHUNK)
    def _(r):
        ones_v[r, :] = unit

    rows = pl.ds(s * RPT, RPT)
    pltpu.sync_copy(zrows_hbm.at[rows], acc_sh.at[rows])
    pltpu.sync_copy(dst_hbm.at[wid], idx_v)
    plsc.subcore_barrier()

    @pl.loop(0, K_CHUNKS)
    def _(j):
        pltpu.sync_copy(ones_v, acc_sh.at[idx_v.at[j]], add=True)

    plsc.subcore_barrier()
    pltpu.sync_copy(acc_sh.at[rows], out_hbm.at[c, rows])


NBUF = 3


def _make_agg_kernel(feat):
    @functools.partial(
        pl.kernel,
        out_type=jax.ShapeDtypeStruct((NC, N_PAD, feat), jnp.float32),
        mesh=_MESH,
        scratch_types=[
            pltpu.VMEM((K_CHUNKS, CHUNK), jnp.int32),
            pltpu.VMEM((K_CHUNKS, CHUNK), jnp.int32),
            [pltpu.VMEM((CHUNK, feat), jnp.float32) for _ in range(NBUF)],
            pltpu.VMEM_SHARED((N_PAD, feat), jnp.float32),
            pltpu.VMEM_SHARED((N_PAD, feat), jnp.float32),
            [pltpu.SemaphoreType.DMA for _ in range(NBUF)],
            [pltpu.SemaphoreType.DMA for _ in range(NBUF)],
        ],
        compiler_params=_SC_PARAMS,
    )
    def agg(hs_hbm, src_hbm, dst_hbm, zrows_hbm, out_hbm,
            srcv, dstv, bufs, hs_sh, acc_sh, gsems, ssems):
        c = lax.axis_index("c")
        s = lax.axis_index("s")
        wid = c * NS + s
        rows = pl.ds(s * RPT, RPT)
        pltpu.sync_copy(hs_hbm.at[rows], hs_sh.at[rows])
        pltpu.sync_copy(zrows_hbm.at[rows], acc_sh.at[rows])
        pltpu.sync_copy(src_hbm.at[wid], srcv)
        pltpu.sync_copy(dst_hbm.at[wid], dstv)
        plsc.subcore_barrier()

        for b in range(NBUF):
            pltpu.async_copy(hs_sh.at[srcv.at[b]], bufs[b], gsems[b])

        @pl.loop(0, K_CHUNKS, step=NBUF)
        def _(g):
            for b in range(NBUF):
                j = g + b

                @pl.when(j < K_CHUNKS)
                def _():
                    pltpu.make_async_copy(
                        hs_sh.at[srcv.at[j]], bufs[b], gsems[b]).wait()
                    pltpu.async_copy(
                        bufs[b], acc_sh.at[dstv.at[j]], ssems[b], add=True)

            for b in range(NBUF):
                j = g + b

                @pl.when(j + NBUF < K_CHUNKS)
                def _():
                    pltpu.make_async_copy(
                        bufs[b], acc_sh.at[dstv.at[j]], ssems[b]).wait()
                    pltpu.async_copy(
                        hs_sh.at[srcv.at[j + NBUF]], bufs[b], gsems[b])

        for b in range(NBUF):
            jlast = ((K_CHUNKS - 1 - b) // NBUF) * NBUF + b
            pltpu.make_async_copy(
                bufs[b], acc_sh.at[dstv.at[jlast]], ssems[b]).wait()

        plsc.subcore_barrier()
        pltpu.sync_copy(acc_sh.at[rows], out_hbm.at[c, rows])

    return agg


_agg64 = _make_agg_kernel(64)


def _dis_from(degp_ref):
    return lax.rsqrt(degp_ref[0, :, 0:1] + degp_ref[1, :, 0:1] + 1.0)


def _mm1_body(x_ref, w_ref, degp_ref, o_ref):
    dis = _dis_from(degp_ref)
    m = jnp.dot(x_ref[...], w_ref[...], preferred_element_type=jnp.float32)
    o_ref[...] = m * dis


def _mid_body(p_ref, hs1_ref, degp_ref, w2_ref, b1_ref, oa_ref, ob_ref):
    dis = _dis_from(degp_ref)
    agg = p_ref[0] + p_ref[1] + hs1_ref[...]
    h2 = jnp.maximum(agg * dis + b1_ref[...], 0.0)
    hs2 = jnp.dot(h2, w2_ref[...], preferred_element_type=jnp.float32) * dis
    oa_ref[...] = hs2[:, :64]
    ob_ref[...] = hs2[:, 64:]


def _out_body(qa_ref, qb_ref, hs2a_ref, hs2b_ref, degp_ref, b2_ref, o_ref):
    dis = _dis_from(degp_ref)
    o_ref[:, :64] = (qa_ref[0] + qa_ref[1] + hs2a_ref[...]) * dis + b2_ref[:, :64]
    o_ref[:, 64:] = (qb_ref[0] + qb_ref[1] + hs2b_ref[...]) * dis + b2_ref[:, 64:]


def _row_spec(f):
    return pl.BlockSpec((BT, f), lambda i: (i, 0))


def _part_spec(f):
    return pl.BlockSpec((NC, BT, f), lambda i: (0, i, 0))


def _full_spec(shape):
    return pl.BlockSpec(shape, lambda i: tuple(0 for _ in shape))


_GRID = (N_PAD // BT,)

_mm1 = pl.pallas_call(
    _mm1_body,
    grid=_GRID,
    in_specs=[_row_spec(128), _full_spec((128, 64)), _part_spec(16)],
    out_specs=_row_spec(64),
    out_shape=jax.ShapeDtypeStruct((N_PAD, 64), jnp.float32),
)

_mid = pl.pallas_call(
    _mid_body,
    grid=_GRID,
    in_specs=[_part_spec(64), _row_spec(64), _part_spec(16),
              _full_spec((64, 128)), _full_spec((1, 64))],
    out_specs=[_row_spec(64), _row_spec(64)],
    out_shape=[jax.ShapeDtypeStruct((N_PAD, 64), jnp.float32)] * 2,
)

_out = pl.pallas_call(
    _out_body,
    grid=_GRID,
    in_specs=[_part_spec(64), _part_spec(64), _row_spec(64), _row_spec(64),
              _part_spec(16), _full_spec((1, 128))],
    out_specs=_row_spec(128),
    out_shape=jax.ShapeDtypeStruct((N_PAD, 128), jnp.float32),
)


def kernel(x, edge_index, W1, b1, W2, b2):
    ei = edge_index.astype(jnp.int32)
    pad_len = EPW_PAD * NW - E_EDGES
    pad = jnp.full((pad_len,), N_NODES, jnp.int32)
    src = jnp.concatenate([ei[0], pad]).reshape(NW, K_CHUNKS, CHUNK)
    dst = jnp.concatenate([ei[1], pad]).reshape(NW, K_CHUNKS, CHUNK)

    x_ext = jnp.zeros((N_PAD, 128), jnp.float32).at[:N_NODES].set(x)
    z16 = jnp.zeros((N_PAD, 16), jnp.float32)
    z64 = jnp.zeros((N_PAD, 64), jnp.float32)

    degp = _deg_kernel(dst, z16)

    hs1 = _mm1(x_ext, W1, degp)
    p = _agg64(hs1, src, dst, z64)
    hs2a, hs2b = _mid(p, hs1, degp, W2, b1.reshape(1, 64))
    qa = _agg64(hs2a, src, dst, z64)
    qb = _agg64(hs2b, src, dst, z64)
    out = _out(qa, qb, hs2a, hs2b, degp, b2.reshape(1, 128))
    return out[:N_NODES]


# speedup vs baseline: 1.1308x; 1.1308x over previous
import functools

import jax
import jax.numpy as jnp
from jax import lax
from jax.experimental import pallas as pl
from jax.experimental.pallas import tpu as pltpu
from jax.experimental.pallas import tpu_sc as plsc

N_NODES = 10000
N_PAD = 10240
E_EDGES = 320000
NC, NS = 2, 16
NW = NC * NS
CHUNK = 128
EPW = E_EDGES // NW
K_CHUNKS = -(-EPW // CHUNK)
EPW_PAD = K_CHUNKS * CHUNK
RPT = N_PAD // NS
BT = 1024

_MESH = plsc.VectorSubcoreMesh(core_axis_name="c", subcore_axis_name="s")
_SC_PARAMS = pltpu.CompilerParams(use_tc_tiling_on_sc=False)


@functools.partial(
    pl.kernel,
    out_type=jax.ShapeDtypeStruct((NC, N_PAD, 16), jnp.float32),
    mesh=_MESH,
    scratch_types=[
        pltpu.VMEM((K_CHUNKS, CHUNK), jnp.int32),
        pltpu.VMEM((CHUNK, 16), jnp.float32),
        pltpu.VMEM_SHARED((N_PAD, 16), jnp.float32),
        pltpu.SemaphoreType.DMA,
    ],
    compiler_params=_SC_PARAMS,
)
def _deg_kernel(dst_hbm, zrows_hbm, out_hbm, idx_v, ones_v, acc_sh, sem):
    c = lax.axis_index("c")
    s = lax.axis_index("s")
    wid = c * NS + s
    lane = lax.iota(jnp.int32, 16)
    unit = jnp.where(lane == 0, jnp.float32(1.0), jnp.float32(0.0))

    @pl.loop(0, CHUNK)
    def _(r):
        ones_v[r, :] = unit

    rows = pl.ds(s * RPT, RPT)
    pltpu.sync_copy(zrows_hbm.at[rows], acc_sh.at[rows])
    pltpu.sync_copy(dst_hbm.at[wid], idx_v)
    plsc.subcore_barrier()

    @pl.loop(0, K_CHUNKS, step=8)
    def _(g):
        for b in range(8):
            @pl.when(g + b < K_CHUNKS)
            def _():
                pltpu.async_copy(ones_v, acc_sh.at[idx_v.at[g + b]], sem,
                                 add=True)
        for b in range(8):
            @pl.when(g + b < K_CHUNKS)
            def _():
                pltpu.make_async_copy(
                    ones_v, acc_sh.at[idx_v.at[g + b]], sem).wait()

    plsc.subcore_barrier()
    pltpu.sync_copy(acc_sh.at[rows], out_hbm.at[c, rows])


NBUF = 2


def _agg_pipeline(hs_sh, acc_sh, srcv, dstv, bufs, gsems, ssems):
    for b in range(NBUF):
        pltpu.async_copy(hs_sh.at[srcv.at[b]], bufs[b], gsems[b])

    @pl.loop(0, K_CHUNKS, step=NBUF)
    def _(g):
        for b in range(NBUF):
            j = g + b

            @pl.when(j < K_CHUNKS)
            def _():
                pltpu.make_async_copy(
                    hs_sh.at[srcv.at[j]], bufs[b], gsems[b]).wait()
                pltpu.async_copy(
                    bufs[b], acc_sh.at[dstv.at[j]], ssems[b], add=True)

        for b in range(NBUF):
            j = g + b

            @pl.when(j + NBUF < K_CHUNKS)
            def _():
                pltpu.make_async_copy(
                    bufs[b], acc_sh.at[dstv.at[j]], ssems[b]).wait()
                pltpu.async_copy(
                    hs_sh.at[srcv.at[j + NBUF]], bufs[b], gsems[b])

    for b in range(NBUF):
        jlast = ((K_CHUNKS - 1 - b) // NBUF) * NBUF + b
        pltpu.make_async_copy(
            bufs[b], acc_sh.at[dstv.at[jlast]], ssems[b]).wait()


def _make_agg_kernel(feat):
    @functools.partial(
        pl.kernel,
        out_type=jax.ShapeDtypeStruct((NC, N_PAD, feat), jnp.float32),
        mesh=_MESH,
        scratch_types=[
            pltpu.VMEM((K_CHUNKS, CHUNK), jnp.int32),
            pltpu.VMEM((K_CHUNKS, CHUNK), jnp.int32),
            [pltpu.VMEM((CHUNK, feat), jnp.float32) for _ in range(NBUF)],
            pltpu.VMEM_SHARED((N_PAD, feat), jnp.float32),
            pltpu.VMEM_SHARED((N_PAD, feat), jnp.float32),
            [pltpu.SemaphoreType.DMA for _ in range(NBUF)],
            [pltpu.SemaphoreType.DMA for _ in range(NBUF)],
        ],
        compiler_params=_SC_PARAMS,
    )
    def agg(hs_hbm, src_hbm, dst_hbm, zrows_hbm, out_hbm,
            srcv, dstv, bufs, hs_sh, acc_sh, gsems, ssems):
        c = lax.axis_index("c")
        s = lax.axis_index("s")
        wid = c * NS + s
        rows = pl.ds(s * RPT, RPT)
        pltpu.sync_copy(hs_hbm.at[rows], hs_sh.at[rows])
        pltpu.sync_copy(zrows_hbm.at[rows], acc_sh.at[rows])
        pltpu.sync_copy(src_hbm.at[wid], srcv)
        pltpu.sync_copy(dst_hbm.at[wid], dstv)
        plsc.subcore_barrier()
        _agg_pipeline(hs_sh, acc_sh, srcv, dstv, bufs, gsems, ssems)
        plsc.subcore_barrier()
        pltpu.sync_copy(acc_sh.at[rows], out_hbm.at[c, rows])

    return agg


@functools.partial(
    pl.kernel,
    out_type=(jax.ShapeDtypeStruct((NC, N_PAD, 64), jnp.float32),
              jax.ShapeDtypeStruct((NC, N_PAD, 64), jnp.float32)),
    mesh=_MESH,
    scratch_types=[
        pltpu.VMEM((K_CHUNKS, CHUNK), jnp.int32),
        pltpu.VMEM((K_CHUNKS, CHUNK), jnp.int32),
        [pltpu.VMEM((CHUNK, 64), jnp.float32) for _ in range(NBUF)],
        pltpu.VMEM_SHARED((N_PAD, 64), jnp.float32),
        pltpu.VMEM_SHARED((N_PAD, 64), jnp.float32),
        [pltpu.SemaphoreType.DMA for _ in range(NBUF)],
        [pltpu.SemaphoreType.DMA for _ in range(NBUF)],
    ],
    compiler_params=_SC_PARAMS,
)
def _agg2_kernel(ha_hbm, hb_hbm, src_hbm, dst_hbm, zrows_hbm, qa_hbm, qb_hbm,
                 srcv, dstv, bufs, hs_sh, acc_sh, gsems, ssems):
    c = lax.axis_index("c")
    s = lax.axis_index("s")
    wid = c * NS + s
    rows = pl.ds(s * RPT, RPT)
    pltpu.sync_copy(ha_hbm.at[rows], hs_sh.at[rows])
    pltpu.sync_copy(zrows_hbm.at[rows], acc_sh.at[rows])
    pltpu.sync_copy(src_hbm.at[wid], srcv)
    pltpu.sync_copy(dst_hbm.at[wid], dstv)
    plsc.subcore_barrier()
    _agg_pipeline(hs_sh, acc_sh, srcv, dstv, bufs, gsems, ssems)
    plsc.subcore_barrier()
    pltpu.sync_copy(acc_sh.at[rows], qa_hbm.at[c, rows])
    pltpu.sync_copy(hb_hbm.at[rows], hs_sh.at[rows])
    pltpu.sync_copy(zrows_hbm.at[rows], acc_sh.at[rows])
    plsc.subcore_barrier()
    _agg_pipeline(hs_sh, acc_sh, srcv, dstv, bufs, gsems, ssems)
    plsc.subcore_barrier()
    pltpu.sync_copy(acc_sh.at[rows], qb_hbm.at[c, rows])


_agg64 = _make_agg_kernel(64)


def _dis_from(degp_ref):
    return lax.rsqrt(degp_ref[0, :, 0:1] + degp_ref[1, :, 0:1] + 1.0)


def _mm1_body(x_ref, w_ref, degp_ref, o_ref):
    dis = _dis_from(degp_ref)
    m = jnp.dot(x_ref[...], w_ref[...], preferred_element_type=jnp.float32)
    o_ref[...] = m * dis


def _mid_body(p_ref, hs1_ref, degp_ref, w2_ref, b1_ref, oa_ref, ob_ref):
    dis = _dis_from(degp_ref)
    agg = p_ref[0] + p_ref[1] + hs1_ref[...]
    h2 = jnp.maximum(agg * dis + b1_ref[...], 0.0)
    hs2 = jnp.dot(h2, w2_ref[...], preferred_element_type=jnp.float32) * dis
    oa_ref[...] = hs2[:, :64]
    ob_ref[...] = hs2[:, 64:]


def _out_body(qa_ref, qb_ref, hs2a_ref, hs2b_ref, degp_ref, b2_ref, o_ref):
    dis = _dis_from(degp_ref)
    o_ref[:, :64] = (qa_ref[0] + qa_ref[1] + hs2a_ref[...]) * dis + b2_ref[:, :64]
    o_ref[:, 64:] = (qb_ref[0] + qb_ref[1] + hs2b_ref[...]) * dis + b2_ref[:, 64:]


def _row_spec(f):
    return pl.BlockSpec((BT, f), lambda i: (i, 0))


def _part_spec(f):
    return pl.BlockSpec((NC, BT, f), lambda i: (0, i, 0))


def _full_spec(shape):
    return pl.BlockSpec(shape, lambda i: tuple(0 for _ in shape))


_GRID = (N_PAD // BT,)

_mm1 = pl.pallas_call(
    _mm1_body,
    grid=_GRID,
    in_specs=[_row_spec(128), _full_spec((128, 64)), _part_spec(16)],
    out_specs=_row_spec(64),
    out_shape=jax.ShapeDtypeStruct((N_PAD, 64), jnp.float32),
)

_mid = pl.pallas_call(
    _mid_body,
    grid=_GRID,
    in_specs=[_part_spec(64), _row_spec(64), _part_spec(16),
              _full_spec((64, 128)), _full_spec((1, 64))],
    out_specs=[_row_spec(64), _row_spec(64)],
    out_shape=[jax.ShapeDtypeStruct((N_PAD, 64), jnp.float32)] * 2,
)

_out = pl.pallas_call(
    _out_body,
    grid=_GRID,
    in_specs=[_part_spec(64), _part_spec(64), _row_spec(64), _row_spec(64),
              _part_spec(16), _full_spec((1, 128))],
    out_specs=_row_spec(128),
    out_shape=jax.ShapeDtypeStruct((N_PAD, 128), jnp.float32),
)


def kernel(x, edge_index, W1, b1, W2, b2):
    ei = edge_index.astype(jnp.int32)
    pad_len = EPW_PAD * NW - E_EDGES
    pad = jnp.full((pad_len,), N_NODES, jnp.int32)
    src = jnp.concatenate([ei[0], pad]).reshape(NW, K_CHUNKS, CHUNK)
    dst = jnp.concatenate([ei[1], pad]).reshape(NW, K_CHUNKS, CHUNK)

    x_ext = jnp.zeros((N_PAD, 128), jnp.float32).at[:N_NODES].set(x)
    z16 = jnp.zeros((N_PAD, 16), jnp.float32)
    z64 = jnp.zeros((N_PAD, 64), jnp.float32)

    degp = _deg_kernel(dst, z16)

    hs1 = _mm1(x_ext, W1, degp)
    p = _agg64(hs1, src, dst, z64)
    hs2a, hs2b = _mid(p, hs1, degp, W2, b1.reshape(1, 64))
    qa, qb = _agg2_kernel(hs2a, hs2b, src, dst, z64)
    out = _out(qa, qb, hs2a, hs2b, degp, b2.reshape(1, 128))
    return out[:N_NODES]


# speedup vs baseline: 1.1314x; 1.0006x over previous
import functools

import jax
import jax.numpy as jnp
from jax import lax
from jax.experimental import pallas as pl
from jax.experimental.pallas import tpu as pltpu
from jax.experimental.pallas import tpu_sc as plsc

N_NODES = 10000
N_PAD = 10240
E_EDGES = 320000
NC, NS = 2, 16
NW = NC * NS
CHUNK = 128
EPW = E_EDGES // NW
K_CHUNKS = -(-EPW // CHUNK)
EPW_PAD = K_CHUNKS * CHUNK
RPT = N_PAD // NS
BT = 1024

_MESH = plsc.VectorSubcoreMesh(core_axis_name="c", subcore_axis_name="s")
_SC_PARAMS = pltpu.CompilerParams(use_tc_tiling_on_sc=False)


@functools.partial(
    pl.kernel,
    out_type=jax.ShapeDtypeStruct((NC, N_PAD, 16), jnp.float32),
    mesh=_MESH,
    scratch_types=[
        pltpu.VMEM((K_CHUNKS, CHUNK), jnp.int32),
        pltpu.VMEM((CHUNK, 16), jnp.float32),
        pltpu.VMEM_SHARED((N_PAD, 16), jnp.float32),
        pltpu.SemaphoreType.DMA,
    ],
    compiler_params=_SC_PARAMS,
)
def _deg_kernel(dst_hbm, zrows_hbm, out_hbm, idx_v, ones_v, acc_sh, sem):
    c = lax.axis_index("c")
    s = lax.axis_index("s")
    wid = c * NS + s
    lane = lax.iota(jnp.int32, 16)
    unit = jnp.where(lane == 0, jnp.float32(1.0), jnp.float32(0.0))

    @pl.loop(0, CHUNK)
    def _(r):
        ones_v[r, :] = unit

    rows = pl.ds(s * RPT, RPT)
    pltpu.sync_copy(zrows_hbm.at[rows], acc_sh.at[rows])
    pltpu.sync_copy(dst_hbm.at[wid], idx_v)
    plsc.subcore_barrier()

    @pl.loop(0, K_CHUNKS, step=8)
    def _(g):
        for b in range(8):
            @pl.when(g + b < K_CHUNKS)
            def _():
                pltpu.async_copy(ones_v, acc_sh.at[idx_v.at[g + b]], sem,
                                 add=True)
        for b in range(8):
            @pl.when(g + b < K_CHUNKS)
            def _():
                pltpu.make_async_copy(
                    ones_v, acc_sh.at[idx_v.at[g + b]], sem).wait()

    plsc.subcore_barrier()
    pltpu.sync_copy(acc_sh.at[rows], out_hbm.at[c, rows])


NBUF = 2


def _agg_pipeline(hs_sh, acc_sh, srcv, dstv, bufs, gsems, ssems):
    for b in range(NBUF):
        pltpu.async_copy(hs_sh.at[srcv.at[b]], bufs[b], gsems[b])

    @pl.loop(0, K_CHUNKS, step=NBUF)
    def _(g):
        for b in range(NBUF):
            j = g + b

            @pl.when(j < K_CHUNKS)
            def _():
                pltpu.make_async_copy(
                    hs_sh.at[srcv.at[j]], bufs[b], gsems[b]).wait()
                pltpu.async_copy(
                    bufs[b], acc_sh.at[dstv.at[j]], ssems[b], add=True)

        for b in range(NBUF):
            j = g + b

            @pl.when(j + NBUF < K_CHUNKS)
            def _():
                pltpu.make_async_copy(
                    bufs[b], acc_sh.at[dstv.at[j]], ssems[b]).wait()
                pltpu.async_copy(
                    hs_sh.at[srcv.at[j + NBUF]], bufs[b], gsems[b])

    for b in range(NBUF):
        jlast = ((K_CHUNKS - 1 - b) // NBUF) * NBUF + b
        pltpu.make_async_copy(
            bufs[b], acc_sh.at[dstv.at[jlast]], ssems[b]).wait()


def _make_agg_kernel(feat):
    @functools.partial(
        pl.kernel,
        out_type=jax.ShapeDtypeStruct((NC, N_PAD, feat), jnp.float32),
        mesh=_MESH,
        scratch_types=[
            pltpu.VMEM((K_CHUNKS, CHUNK), jnp.int32),
            pltpu.VMEM((K_CHUNKS, CHUNK), jnp.int32),
            [pltpu.VMEM((CHUNK, feat), jnp.float32) for _ in range(NBUF)],
            pltpu.VMEM_SHARED((N_PAD, feat), jnp.float32),
            pltpu.VMEM_SHARED((N_PAD, feat), jnp.float32),
            [pltpu.SemaphoreType.DMA for _ in range(NBUF)],
            [pltpu.SemaphoreType.DMA for _ in range(NBUF)],
        ],
        compiler_params=_SC_PARAMS,
    )
    def agg(hs_hbm, src_hbm, dst_hbm, zrows_hbm, out_hbm,
            srcv, dstv, bufs, hs_sh, acc_sh, gsems, ssems):
        c = lax.axis_index("c")
        s = lax.axis_index("s")
        wid = c * NS + s
        rows = pl.ds(s * RPT, RPT)
        pltpu.sync_copy(hs_hbm.at[rows], hs_sh.at[rows])
        pltpu.sync_copy(zrows_hbm.at[rows], acc_sh.at[rows])
        pltpu.sync_copy(src_hbm.at[wid], srcv)
        pltpu.sync_copy(dst_hbm.at[wid], dstv)
        plsc.subcore_barrier()
        _agg_pipeline(hs_sh, acc_sh, srcv, dstv, bufs, gsems, ssems)
        plsc.subcore_barrier()
        pltpu.sync_copy(acc_sh.at[rows], out_hbm.at[c, rows])

    return agg


@functools.partial(
    pl.kernel,
    out_type=(jax.ShapeDtypeStruct((NC, N_PAD, 64), jnp.float32),
              jax.ShapeDtypeStruct((NC, N_PAD, 64), jnp.float32)),
    mesh=_MESH,
    scratch_types=[
        pltpu.VMEM((K_CHUNKS, CHUNK), jnp.int32),
        pltpu.VMEM((K_CHUNKS, CHUNK), jnp.int32),
        [pltpu.VMEM((CHUNK, 64), jnp.float32) for _ in range(NBUF)],
        pltpu.VMEM_SHARED((N_PAD, 64), jnp.float32),
        pltpu.VMEM_SHARED((N_PAD, 64), jnp.float32),
        [pltpu.SemaphoreType.DMA for _ in range(NBUF)],
        [pltpu.SemaphoreType.DMA for _ in range(NBUF)],
    ],
    compiler_params=_SC_PARAMS,
)
def _agg2_kernel(ha_hbm, hb_hbm, src_hbm, dst_hbm, zrows_hbm, qa_hbm, qb_hbm,
                 srcv, dstv, bufs, hs_sh, acc_sh, gsems, ssems):
    c = lax.axis_index("c")
    s = lax.axis_index("s")
    wid = c * NS + s
    rows = pl.ds(s * RPT, RPT)
    pltpu.sync_copy(ha_hbm.at[rows], hs_sh.at[rows])
    pltpu.sync_copy(zrows_hbm.at[rows], acc_sh.at[rows])
    pltpu.sync_copy(src_hbm.at[wid], srcv)
    pltpu.sync_copy(dst_hbm.at[wid], dstv)
    plsc.subcore_barrier()
    _agg_pipeline(hs_sh, acc_sh, srcv, dstv, bufs, gsems, ssems)
    plsc.subcore_barrier()
    pltpu.sync_copy(acc_sh.at[rows], qa_hbm.at[c, rows])
    pltpu.sync_copy(hb_hbm.at[rows], hs_sh.at[rows])
    pltpu.sync_copy(zrows_hbm.at[rows], acc_sh.at[rows])
    plsc.subcore_barrier()
    _agg_pipeline(hs_sh, acc_sh, srcv, dstv, bufs, gsems, ssems)
    plsc.subcore_barrier()
    pltpu.sync_copy(acc_sh.at[rows], qb_hbm.at[c, rows])


_agg64 = _make_agg_kernel(64)


def _dis_from(degp_ref):
    return lax.rsqrt(degp_ref[0, :, 0:1] + degp_ref[1, :, 0:1] + 1.0)


def _mmul_body(x_ref, w_ref, o_ref):
    o_ref[...] = jnp.dot(x_ref[...], w_ref[...],
                         preferred_element_type=jnp.float32)


def _scale_body(m_ref, degp_ref, o_ref):
    o_ref[...] = m_ref[...] * _dis_from(degp_ref)


def _mid_body(p_ref, hs1_ref, degp_ref, w2_ref, b1_ref, oa_ref, ob_ref):
    dis = _dis_from(degp_ref)
    agg = p_ref[0] + p_ref[1] + hs1_ref[...]
    h2 = jnp.maximum(agg * dis + b1_ref[...], 0.0)
    hs2 = jnp.dot(h2, w2_ref[...], preferred_element_type=jnp.float32) * dis
    oa_ref[...] = hs2[:, :64]
    ob_ref[...] = hs2[:, 64:]


def _out_body(qa_ref, qb_ref, hs2a_ref, hs2b_ref, degp_ref, b2_ref, o_ref):
    dis = _dis_from(degp_ref)
    o_ref[:, :64] = (qa_ref[0] + qa_ref[1] + hs2a_ref[...]) * dis + b2_ref[:, :64]
    o_ref[:, 64:] = (qb_ref[0] + qb_ref[1] + hs2b_ref[...]) * dis + b2_ref[:, 64:]


def _row_spec(f):
    return pl.BlockSpec((BT, f), lambda i: (i, 0))


def _part_spec(f):
    return pl.BlockSpec((NC, BT, f), lambda i: (0, i, 0))


def _full_spec(shape):
    return pl.BlockSpec(shape, lambda i: tuple(0 for _ in shape))


_GRID = (N_PAD // BT,)

_mmul = pl.pallas_call(
    _mmul_body,
    grid=_GRID,
    in_specs=[_row_spec(128), _full_spec((128, 64))],
    out_specs=_row_spec(64),
    out_shape=jax.ShapeDtypeStruct((N_PAD, 64), jnp.float32),
)

_scale = pl.pallas_call(
    _scale_body,
    grid=_GRID,
    in_specs=[_row_spec(64), _part_spec(16)],
    out_specs=_row_spec(64),
    out_shape=jax.ShapeDtypeStruct((N_PAD, 64), jnp.float32),
)

_mid = pl.pallas_call(
    _mid_body,
    grid=_GRID,
    in_specs=[_part_spec(64), _row_spec(64), _part_spec(16),
              _full_spec((64, 128)), _full_spec((1, 64))],
    out_specs=[_row_spec(64), _row_spec(64)],
    out_shape=[jax.ShapeDtypeStruct((N_PAD, 64), jnp.float32)] * 2,
)

_out = pl.pallas_call(
    _out_body,
    grid=_GRID,
    in_specs=[_part_spec(64), _part_spec(64), _row_spec(64), _row_spec(64),
              _part_spec(16), _full_spec((1, 128))],
    out_specs=_row_spec(128),
    out_shape=jax.ShapeDtypeStruct((N_PAD, 128), jnp.float32),
)


def kernel(x, edge_index, W1, b1, W2, b2):
    ei = edge_index.astype(jnp.int32)
    pad_len = EPW_PAD * NW - E_EDGES
    pad = jnp.full((pad_len,), N_NODES, jnp.int32)
    src = jnp.concatenate([ei[0], pad]).reshape(NW, K_CHUNKS, CHUNK)
    dst = jnp.concatenate([ei[1], pad]).reshape(NW, K_CHUNKS, CHUNK)

    x_ext = jnp.zeros((N_PAD, 128), jnp.float32).at[:N_NODES].set(x)
    z16 = jnp.zeros((N_PAD, 16), jnp.float32)
    z64 = jnp.zeros((N_PAD, 64), jnp.float32)

    m1 = _mmul(x_ext, W1)
    degp = _deg_kernel(dst, z16)
    hs1 = _scale(m1, degp)
    p = _agg64(hs1, src, dst, z64)
    hs2a, hs2b = _mid(p, hs1, degp, W2, b1.reshape(1, 64))
    qa, qb = _agg2_kernel(hs2a, hs2b, src, dst, z64)
    out = _out(qa, qb, hs2a, hs2b, degp, b2.reshape(1, 128))
    return out[:N_NODES]
